# baseline (speedup 1.0000x reference)
import functools

import jax
import jax.numpy as jnp
from jax import lax
from jax.experimental import pallas as pl
from jax.experimental.pallas import tpu as pltpu
from jax.experimental.pallas import tpu_sc as plsc

EMBED_DIM = 64
NC = 2
NS = 16
NW = NC * NS
ROWS, SEQ = 4096, 50
SEQ_AL = 56
SRC_PAD = 128
R_PER_W = ROWS // NW
G = 8
NGROUP = R_PER_W // G

_mesh = plsc.VectorSubcoreMesh(core_axis_name="c", subcore_axis_name="s")


@functools.partial(
    pl.kernel,
    mesh=_mesh,
    out_type=jax.ShapeDtypeStruct((ROWS, SEQ, EMBED_DIM), jnp.float32),
    scratch_types=[
        pltpu.VMEM((R_PER_W, SRC_PAD), jnp.int32),
        pltpu.VMEM((2, G, SEQ_AL, EMBED_DIM), jnp.float32),
        pltpu.SemaphoreType.DMA,
        pltpu.SemaphoreType.DMA,
        pltpu.SemaphoreType.DMA,
        pltpu.SemaphoreType.DMA,
    ],
    compiler_params=pltpu.CompilerParams(use_tc_tiling_on_sc=False),
)
def _embed(src_hbm, table_hbm, out_hbm, idx_v, rows_v, gsem0, gsem1, esem0, esem1):
    wid = lax.axis_index("s") * NC + lax.axis_index("c")
    rbase = wid * R_PER_W

    pltpu.sync_copy(src_hbm.at[pl.ds(rbase, R_PER_W)], idx_v)

    gsems = (gsem0, gsem1)
    esems = (esem0, esem1)

    def group_start(c, buf):
        for j in range(G):
            pltpu.async_copy(
                table_hbm.at[idx_v.at[c * G + j, pl.ds(0, SEQ_AL)]],
                rows_v.at[buf, j],
                gsems[buf],
            )

    def group_wait(buf):
        for j in range(G):
            pltpu.make_async_copy(
                table_hbm.at[pl.ds(0, SEQ_AL)], rows_v.at[buf, j], gsems[buf]
            ).wait()

    def emit(c, buf):
        for j in range(G):
            pltpu.async_copy(
                rows_v.at[buf, pl.ds(j, 1), pl.ds(0, SEQ)],
                out_hbm.at[pl.ds(rbase + c * G + j, 1)],
                esems[buf],
            )
        pltpu.make_async_copy(
            out_hbm.at[pl.ds(0, G)],
            rows_v.at[buf, :, pl.ds(0, SEQ)],
            esems[buf],
        ).wait()

    group_start(0, 0)

    @pl.loop(0, NGROUP - 2, step=2)
    def _(g):
        group_start(g + 1, 1)
        group_wait(0)
        emit(g, 0)
        group_start(g + 2, 0)
        group_wait(1)
        emit(g + 1, 1)

    group_start(NGROUP - 1, 1)
    group_wait(0)
    emit(NGROUP - 2, 0)
    group_wait(1)
    emit(NGROUP - 1, 1)


def kernel(src, table):
    src_p = jnp.pad(src.astype(jnp.int32), ((0, 0), (0, SRC_PAD - SEQ)))
    return _embed(src_p, table)
